# baseline (speedup 1.0000x reference)
import functools

import jax
import jax.numpy as jnp
from jax import lax
from jax.experimental import pallas as pl
from jax.experimental.pallas import tpu as pltpu
from jax.experimental.pallas import tpu_sc as plsc

GAMMA = 12.0
L = 16
NC = 2
NS = 16
NW = NC * NS


def _make_sc_kernel(dim, batch, neg):
    assert batch % NW == 0 and dim % L == 0 and neg % 2 == 0
    half = neg // 2
    b_per_w = batch // NW
    nj = dim // L
    nfull = neg // L

    mesh = plsc.VectorSubcoreMesh(
        core_axis_name="c", subcore_axis_name="s",
        num_cores=NC, num_subcores=NS)

    @functools.partial(
        pl.kernel,
        out_type=jax.ShapeDtypeStruct((batch, neg), jnp.float32),
        mesh=mesh,
        compiler_params=pltpu.CompilerParams(needs_layout_passes=False),
        scratch_types=[
            pltpu.VMEM((b_per_w,), jnp.int32),
            pltpu.VMEM((b_per_w,), jnp.int32),
            pltpu.VMEM((2 * b_per_w, half), jnp.int32),
            pltpu.VMEM((b_per_w, dim), jnp.float32),
            pltpu.VMEM((b_per_w, dim), jnp.float32),
            pltpu.VMEM((3, neg, dim), jnp.float32),
            pltpu.VMEM(((nfull + 1) * L * L,), jnp.float32),
            pltpu.VMEM((b_per_w, neg), jnp.float32),
            pltpu.SemaphoreType.DMA,
            pltpu.SemaphoreType.DMA,
            pltpu.SemaphoreType.DMA,
            pltpu.SemaphoreType.DMA,
        ],
    )
    def score_kernel(ent_hbm, rel_hbm, hidx_hbm, ridx_hbm, tidx_hbm, out_hbm,
                     hidx_v, ridx_v, tidx_v, hr_v, rrows_v, tbuf_v, tr_v,
                     scores_v, sem_h, sem0, sem1, sem2):
        sems = [sem0, sem1, sem2]
        wid = lax.axis_index("s") * NC + lax.axis_index("c")
        base = wid * b_per_w

        def fire(r, slot, sem):
            pltpu.async_copy(ent_hbm.at[tidx_v.at[2 * r]],
                             tbuf_v.at[slot, pl.ds(0, half)], sem)
            pltpu.async_copy(ent_hbm.at[tidx_v.at[2 * r + 1]],
                             tbuf_v.at[slot, pl.ds(half, half)], sem)

        def wait(r, slot, sem):
            pltpu.make_async_copy(ent_hbm.at[tidx_v.at[2 * r]],
                                  tbuf_v.at[slot, pl.ds(0, half)], sem).wait()
            pltpu.make_async_copy(ent_hbm.at[tidx_v.at[2 * r + 1]],
                                  tbuf_v.at[slot, pl.ds(half, half)],
                                  sem).wait()

        pltpu.sync_copy(tidx_hbm.at[pl.ds(wid * 2 * b_per_w, 2 * b_per_w)],
                        tidx_v)
        fire(0, 0, sem0)
        fire(1, 1, sem1)

        pltpu.sync_copy(hidx_hbm.at[pl.ds(base, b_per_w)], hidx_v)
        pltpu.sync_copy(ridx_hbm.at[pl.ds(base, b_per_w)], ridx_v)
        pltpu.async_copy(ent_hbm.at[hidx_v], hr_v, sem_h).wait()
        pltpu.async_copy(rel_hbm.at[ridx_v], rrows_v, sem_h).wait()

        def hr_body(b, _):
            for j in range(nj):
                s = pl.ds(j * L, L)
                hr_v[b, s] = hr_v[b, s] + rrows_v[b, s]
            return 0
        lax.fori_loop(0, b_per_w, hr_body, 0)

        iota_l = lax.iota(jnp.int32, L) * L

        def compute_row(b, slot):
            hr = [hr_v[b, pl.ds(j * L, L)] for j in range(nj)]

            def group_16(col, tr0):
                @plsc.parallel_loop(0, L, unroll=2)
                def pair(p):
                    t = [tbuf_v[slot, col + p, pl.ds(j * L, L)]
                         for j in range(nj)]
                    d = [jnp.abs(hr[j] - t[j]) for j in range(nj)]
                    while len(d) > 1:
                        d = [d[i] + d[i + 1] for i in range(0, len(d) - 1, 2)] \
                            + ([d[-1]] if len(d) % 2 else [])
                    plsc.store_scatter(tr_v, [iota_l + (tr0 + p)], d[0])
                acc = tr_v[pl.ds(tr0, L)]
                for j in range(1, L):
                    acc = acc + tr_v[pl.ds(tr0 + j * L, L)]
                scores_v[b, pl.ds(col, L)] = GAMMA - acc

            @plsc.parallel_loop(0, nfull + 1)
            def group(g):
                group_16(jnp.minimum(g * L, neg - L), g * (L * L))

        def row_iter(r, _):
            q = lax.rem(r, 3)
            lax.switch(q, [lambda: wait(r, 0, sem0),
                           lambda: wait(r, 1, sem1),
                           lambda: wait(r, 2, sem2)])

            @pl.when(r + 2 < b_per_w)
            def _():
                nxt = r + 2
                lax.switch(lax.rem(nxt, 3),
                           [lambda: fire(nxt, 0, sem0),
                            lambda: fire(nxt, 1, sem1),
                            lambda: fire(nxt, 2, sem2)])
            compute_row(r, q)
            return 0
        lax.fori_loop(0, b_per_w, row_iter, 0)

        pltpu.sync_copy(scores_v, out_hbm.at[pl.ds(base, b_per_w)])

    return score_kernel


def kernel(entity_embedding, relation_embedding, head_part, tail_part):
    dim = entity_embedding.shape[1]
    batch, neg = tail_part.shape
    hidx = head_part[:, 0]
    ridx = head_part[:, 1]
    tidx = tail_part.reshape(2 * batch, neg // 2)
    fn = _make_sc_kernel(dim, batch, neg)
    return fn(entity_embedding, relation_embedding, hidx, ridx, tidx)


# speedup vs baseline: 1.0178x; 1.0178x over previous
import functools

import jax
import jax.numpy as jnp
from jax import lax
from jax.experimental import pallas as pl
from jax.experimental.pallas import tpu as pltpu
from jax.experimental.pallas import tpu_sc as plsc

GAMMA = 12.0
L = 16
NC = 2
NS = 16
NW = NC * NS


def _make_sc_kernel(dim, batch, neg):
    assert batch % NW == 0 and dim % L == 0 and neg % 2 == 0
    half = neg // 2
    b_per_w = batch // NW
    nj = dim // L
    nfull = neg // L

    mesh = plsc.VectorSubcoreMesh(
        core_axis_name="c", subcore_axis_name="s",
        num_cores=NC, num_subcores=NS)

    @functools.partial(
        pl.kernel,
        out_type=jax.ShapeDtypeStruct((batch, neg), jnp.float32),
        mesh=mesh,
        compiler_params=pltpu.CompilerParams(needs_layout_passes=False),
        scratch_types=[
            pltpu.VMEM((b_per_w,), jnp.int32),
            pltpu.VMEM((b_per_w,), jnp.int32),
            pltpu.VMEM((2 * b_per_w, half), jnp.int32),
            pltpu.VMEM((b_per_w, dim), jnp.float32),
            pltpu.VMEM((b_per_w, dim), jnp.float32),
            pltpu.VMEM((3, neg, dim), jnp.float32),
            pltpu.VMEM(((nfull + 1) * L * L,), jnp.float32),
            pltpu.VMEM((b_per_w, neg), jnp.float32),
            pltpu.SemaphoreType.DMA,
            pltpu.SemaphoreType.DMA,
            pltpu.SemaphoreType.DMA,
            pltpu.SemaphoreType.DMA,
        ],
    )
    def score_kernel(ent_hbm, rel_hbm, hidx_hbm, ridx_hbm, tidx_hbm, out_hbm,
                     hidx_v, ridx_v, tidx_v, hr_v, rrows_v, tbuf_v, tr_v,
                     scores_v, sem_h, sem0, sem1, sem2):
        sems = [sem0, sem1, sem2]
        wid = lax.axis_index("s") * NC + lax.axis_index("c")
        base = wid * b_per_w

        def fire(r, slot, sem):
            pltpu.async_copy(ent_hbm.at[tidx_v.at[2 * r]],
                             tbuf_v.at[slot, pl.ds(0, half)], sem)
            pltpu.async_copy(ent_hbm.at[tidx_v.at[2 * r + 1]],
                             tbuf_v.at[slot, pl.ds(half, half)], sem)

        def wait(r, slot, sem):
            pltpu.make_async_copy(ent_hbm.at[tidx_v.at[2 * r]],
                                  tbuf_v.at[slot, pl.ds(0, half)], sem).wait()
            pltpu.make_async_copy(ent_hbm.at[tidx_v.at[2 * r + 1]],
                                  tbuf_v.at[slot, pl.ds(half, half)],
                                  sem).wait()

        pltpu.sync_copy(tidx_hbm.at[pl.ds(wid * 2 * b_per_w, 2 * b_per_w)],
                        tidx_v)
        fire(0, 0, sem0)
        fire(1, 1, sem1)

        pltpu.sync_copy(hidx_hbm.at[pl.ds(base, b_per_w)], hidx_v)
        pltpu.sync_copy(ridx_hbm.at[pl.ds(base, b_per_w)], ridx_v)
        pltpu.async_copy(ent_hbm.at[hidx_v], hr_v, sem_h).wait()
        pltpu.async_copy(rel_hbm.at[ridx_v], rrows_v, sem_h).wait()

        def hr_body(b, _):
            for j in range(nj):
                s = pl.ds(j * L, L)
                hr_v[b, s] = hr_v[b, s] + rrows_v[b, s]
            return 0
        lax.fori_loop(0, b_per_w, hr_body, 0)

        col_idx = [lax.iota(jnp.int32, L) * L + p for p in range(L)]

        def compute_row(b, slot):
            hr = [hr_v[b, pl.ds(j * L, L)] for j in range(nj)]

            def group_16(col, tr0):
                for p in range(L):
                    t = [tbuf_v[slot, col + p, pl.ds(j * L, L)]
                         for j in range(nj)]
                    d = [jnp.abs(hr[j] - t[j]) for j in range(nj)]
                    while len(d) > 1:
                        d = [d[i] + d[i + 1] for i in range(0, len(d) - 1, 2)] \
                            + ([d[-1]] if len(d) % 2 else [])
                    plsc.store_scatter(tr_v, [tr0 + col_idx[p]], d[0])
                acc = tr_v[pl.ds(tr0, L)]
                for j in range(1, L):
                    acc = acc + tr_v[pl.ds(tr0 + j * L, L)]
                scores_v[b, pl.ds(col, L)] = GAMMA - acc

            @plsc.parallel_loop(0, nfull + 1)
            def group(g):
                group_16(jnp.minimum(g * L, neg - L), g * (L * L))

        def row_iter(r, _):
            q = lax.rem(r, 3)
            lax.switch(q, [lambda: wait(r, 0, sem0),
                           lambda: wait(r, 1, sem1),
                           lambda: wait(r, 2, sem2)])

            @pl.when(r + 2 < b_per_w)
            def _():
                nxt = r + 2
                lax.switch(lax.rem(nxt, 3),
                           [lambda: fire(nxt, 0, sem0),
                            lambda: fire(nxt, 1, sem1),
                            lambda: fire(nxt, 2, sem2)])
            compute_row(r, q)
            return 0
        lax.fori_loop(0, b_per_w, row_iter, 0)

        pltpu.sync_copy(scores_v, out_hbm.at[pl.ds(base, b_per_w)])

    return score_kernel


def kernel(entity_embedding, relation_embedding, head_part, tail_part):
    dim = entity_embedding.shape[1]
    batch, neg = tail_part.shape
    hidx = head_part[:, 0]
    ridx = head_part[:, 1]
    tidx = tail_part.reshape(2 * batch, neg // 2)
    fn = _make_sc_kernel(dim, batch, neg)
    return fn(entity_embedding, relation_embedding, hidx, ridx, tidx)
